# baseline (speedup 1.0000x reference)
import functools

import jax
import jax.numpy as jnp
from jax import lax
from jax.experimental import pallas as pl
from jax.experimental.pallas import tpu as pltpu
from jax.experimental.pallas import tpu_sc as plsc

_D = 32
_B = 16384
_V = 100001
_NC = 2
_H = _B // 2

_mesh = plsc.VectorSubcoreMesh(core_axis_name="c", subcore_axis_name="s")


@functools.partial(
    pl.kernel,
    mesh=_mesh,
    compiler_params=pltpu.CompilerParams(needs_layout_passes=False),
    out_type=jax.ShapeDtypeStruct((_D, _B), jnp.float32),
    scratch_types=[
        pltpu.VMEM((_V,), jnp.float32),
        pltpu.VMEM((_H,), jnp.int32),
        pltpu.VMEM((_H,), jnp.float32),
        pltpu.SemaphoreType.DMA,
        pltpu.SemaphoreType.DMA,
    ],
)
def _gather_kernel(tbl_hbm, idx_hbm, out_hbm, row_v, idx_v, orow_v,
                   rsem, isem):
    d = lax.axis_index("s") * _NC + lax.axis_index("c")
    row_cp = pltpu.async_copy(tbl_hbm.at[d], row_v, rsem)
    idx_cp = pltpu.async_copy(idx_hbm.at[pl.ds(0, _H)], idx_v, isem)
    row_cp.wait()

    def half(h, carry):
        idx_cp.wait()

        @plsc.parallel_loop(0, _H // 16, step=1, unroll=8)
        def _grp(g):
            vec = idx_v[pl.ds(g * 16, 16)]
            orow_v[pl.ds(g * 16, 16)] = plsc.load_gather(row_v, [vec])

        @pl.when(h == 0)
        def _():
            pltpu.async_copy(idx_hbm.at[pl.ds(_H, _H)], idx_v, isem)

        pltpu.sync_copy(orow_v, out_hbm.at[d, pl.ds(h * _H, _H)])
        return carry

    lax.fori_loop(0, 2, half, 0)


def kernel(titles, table):
    out_t = _gather_kernel(table.T, titles.astype(jnp.int32))
    return out_t.T


# speedup vs baseline: 1.0783x; 1.0783x over previous
import functools

import jax
import jax.numpy as jnp
from jax import lax
from jax.experimental import pallas as pl
from jax.experimental.pallas import tpu as pltpu
from jax.experimental.pallas import tpu_sc as plsc

_D = 32
_B = 16384
_V = 100001
_NC = 2
_H = _B // 2

_mesh = plsc.VectorSubcoreMesh(core_axis_name="c", subcore_axis_name="s")


@functools.partial(
    pl.kernel,
    mesh=_mesh,
    compiler_params=pltpu.CompilerParams(needs_layout_passes=False),
    out_type=jax.ShapeDtypeStruct((_D, _B), jnp.float32),
    scratch_types=[
        pltpu.VMEM((_V,), jnp.float32),
        pltpu.VMEM((_H,), jnp.int32),
        pltpu.VMEM((_H,), jnp.int32),
        pltpu.VMEM((_H,), jnp.float32),
        pltpu.SemaphoreType.DMA,
        pltpu.SemaphoreType.DMA,
    ],
)
def _gather_kernel(tbl_hbm, idx_hbm, out_hbm, row_v, idx0_v, idx1_v, orow_v,
                   rsem, isem):
    d = lax.axis_index("s") * _NC + lax.axis_index("c")
    row_cp = pltpu.async_copy(tbl_hbm.at[d], row_v, rsem)
    idx0_cp = pltpu.async_copy(idx_hbm.at[pl.ds(0, _H)], idx0_v, isem)
    idx1_cp = pltpu.async_copy(idx_hbm.at[pl.ds(_H, _H)], idx1_v, isem)
    row_cp.wait()

    def half(h, idx_v):
        @plsc.parallel_loop(0, _H // 16, step=1, unroll=16)
        def _grp(g):
            vec = idx_v[pl.ds(g * 16, 16)]
            orow_v[pl.ds(g * 16, 16)] = plsc.load_gather(row_v, [vec])

        pltpu.sync_copy(orow_v, out_hbm.at[d, pl.ds(h * _H, _H)])

    idx0_cp.wait()
    half(0, idx0_v)
    idx1_cp.wait()
    half(1, idx1_v)


def kernel(titles, table):
    out_t = _gather_kernel(table.T, titles.astype(jnp.int32))
    return out_t.T
